# Initial kernel scaffold; baseline (speedup 1.0000x reference)
import jax
import jax.numpy as jnp
from jax.experimental import pallas as pl


def kernel(nid, k_, q_, v_, t, neighbors, times, w0, b0, w, b, Wk, bk, Wq, bq, Wv, bv):
    raise NotImplementedError("write your pallas kernel here")


# speedup vs baseline: 1.7756x; 1.7756x over previous
import functools
import math

import jax
import jax.numpy as jnp
from jax import lax
from jax.experimental import pallas as pl
from jax.experimental.pallas import tpu as pltpu
from jax.experimental.pallas import tpu_sc as plsc

N = 100000
HIDDEN = 128
T2V_DIM = 64
DEG = 64
LANES = 16
CHUNKS = HIDDEN // LANES


def _sc_body(nbr_hbm, times_hbm, t_hbm, v_hbm, out_hbm,
             idx_v, times_v, t_v, rows_v, acc_v, sem):
    c = lax.axis_index("c")
    s = lax.axis_index("s")

    @pl.when(jnp.logical_and(c == 0, s == 0))
    def _():
        pltpu.sync_copy(nbr_hbm, idx_v)
        pltpu.sync_copy(times_hbm, times_v)
        pltpu.sync_copy(t_hbm, t_v)
        pltpu.async_copy(v_hbm.at[idx_v], rows_v, sem).wait()

        tvec = t_v[...]
        accs = [jnp.zeros((LANES,), jnp.float32) for _ in range(CHUNKS)]
        for g in range(DEG // LANES):
            times16 = times_v[pl.ds(g * LANES, LANES)]
            mvec = jnp.where(times16 <= tvec, 1.0, 0.0)
            for l in range(LANES):
                m = mvec[l]
                j = g * LANES + l
                for k in range(CHUNKS):
                    accs[k] = accs[k] + rows_v[j, pl.ds(k * LANES, LANES)] * m
        for k in range(CHUNKS):
            acc_v[pl.ds(k * LANES, LANES)] = accs[k]
        pltpu.sync_copy(acc_v, out_hbm)


@jax.jit
def _sc_gather_sum(v_, nbr, times, t16):
    mesh = plsc.VectorSubcoreMesh(core_axis_name="c", subcore_axis_name="s")
    return pl.kernel(
        _sc_body,
        out_type=jax.ShapeDtypeStruct((HIDDEN,), jnp.float32),
        mesh=mesh,
        scratch_types=[
            pltpu.VMEM((DEG,), jnp.int32),
            pltpu.VMEM((DEG,), jnp.float32),
            pltpu.VMEM((LANES,), jnp.float32),
            pltpu.VMEM((DEG, HIDDEN), jnp.float32),
            pltpu.VMEM((HIDDEN,), jnp.float32),
            pltpu.SemaphoreType.DMA,
        ],
    )(nbr, times, t16, v_)


def _tc_body(times_col_ref, wfull_ref, bfull_ref, t_ref, Wv_ref, bv_ref,
             out_ref):
    tval = t_ref[0]
    times_col = times_col_ref[...]
    a = times_col * wfull_ref[...] + bfull_ref[...]
    col = lax.broadcasted_iota(jnp.int32, (DEG, T2V_DIM), 1)
    z = jnp.where(col == 0, a, jnp.sin(a))
    m = jnp.where(times_col <= tval, 1.0, 0.0)
    zsum = jnp.sum(z * m, axis=0)
    cnt = jnp.sum(m)
    tv = jnp.sum(Wv_ref[...] * zsum[:, None], axis=0, keepdims=True)
    out_ref[...] = tv + cnt * bv_ref[...]


@jax.jit
def _tc_time_branch(times_col, wfull, bfull, t, Wv, bv_row):
    return pl.pallas_call(
        _tc_body,
        out_shape=jax.ShapeDtypeStruct((1, HIDDEN), jnp.float32),
        in_specs=[
            pl.BlockSpec(memory_space=pltpu.VMEM),
            pl.BlockSpec(memory_space=pltpu.VMEM),
            pl.BlockSpec(memory_space=pltpu.VMEM),
            pl.BlockSpec(memory_space=pltpu.SMEM),
            pl.BlockSpec(memory_space=pltpu.VMEM),
            pl.BlockSpec(memory_space=pltpu.VMEM),
        ],
        out_specs=pl.BlockSpec(memory_space=pltpu.VMEM),
    )(times_col, wfull, bfull, t, Wv, bv_row)


def kernel(nid, k_, q_, v_, t, neighbors, times, w0, b0, w, b,
           Wk, bk, Wq, bq, Wv, bv):
    del nid, k_, q_, Wk, bk, Wq, bq
    nbr = neighbors.astype(jnp.int32)
    t16 = jnp.broadcast_to(t.astype(jnp.float32), (LANES,))
    times_col = times.reshape(DEG, 1)
    wfull = jnp.concatenate([w0.reshape(1, 1), w.reshape(1, T2V_DIM - 1)], axis=1)
    bfull = jnp.concatenate([b0.reshape(1, 1), b.reshape(1, T2V_DIM - 1)], axis=1)
    gsum = _sc_gather_sum(v_, nbr, times, t16)
    tvp = _tc_time_branch(times_col, wfull, bfull, t, Wv, bv.reshape(1, HIDDEN))
    return tvp + gsum[None, :]
